# Initial kernel scaffold; baseline (speedup 1.0000x reference)
import jax
import jax.numpy as jnp
from jax.experimental import pallas as pl


def kernel(inputs, emb_table, W, b):
    raise NotImplementedError("write your pallas kernel here")


# speedup vs baseline: 1.1906x; 1.1906x over previous
import functools

import jax
import jax.numpy as jnp
from jax import lax
from jax.experimental import pallas as pl
from jax.experimental.pallas import tpu as pltpu
from jax.experimental.pallas import tpu_sc as plsc

V = 100000
D = 128
CTX = 16384
NW = 32
ROWS_PER_W = CTX // NW
CHUNK = 128
NCHUNK = ROWS_PER_W // CHUNK
LANES = 16
NVEC = D // LANES

BLK = 8192
NB = (V + BLK - 1) // BLK


def _sc_gather_sum(idx2d, table):
    mesh = plsc.VectorSubcoreMesh(core_axis_name="c", subcore_axis_name="s")

    @functools.partial(
        pl.kernel,
        out_type=jax.ShapeDtypeStruct((NW, D), jnp.float32),
        mesh=mesh,
        scratch_types=[
            pltpu.VMEM((NCHUNK, CHUNK), jnp.int32),
            pltpu.VMEM((NCHUNK, CHUNK, D), jnp.float32),
            pltpu.VMEM((D,), jnp.float32),
            pltpu.SemaphoreType.DMA,
        ],
    )
    def k(idx_hbm, table_hbm, out_hbm, idx_v, rows_v, acc_v, sem):
        wid = lax.axis_index("s") * 2 + lax.axis_index("c")
        pltpu.sync_copy(idx_hbm.at[pl.ds(wid * NCHUNK, NCHUNK)], idx_v)
        copies = [
            pltpu.async_copy(table_hbm.at[idx_v.at[c]], rows_v.at[c], sem)
            for c in range(NCHUNK)
        ]
        for cp in copies:
            cp.wait()

        def outer(acc, c):
            def body(i, acc):
                return tuple(
                    acc[j] + rows_v[c, i, pl.ds(j * LANES, LANES)]
                    for j in range(NVEC)
                )
            return lax.fori_loop(0, CHUNK, body, acc)

        acc = tuple(jnp.zeros((LANES,), jnp.float32) for _ in range(NVEC))
        for c in range(NCHUNK):
            acc = outer(acc, c)
        for j in range(NVEC):
            acc_v[pl.ds(j * LANES, LANES)] = acc[j]
        pltpu.sync_copy(acc_v, out_hbm.at[wid])

    return k(idx2d, table)


def _tc_logits_lse(partials, W, b2d):
    def body(part_ref, w_ref, b_ref, out_ref, lse_ref, m_s, s_s):
        j = pl.program_id(0)

        @pl.when(j == 0)
        def _():
            m_s[0] = -1e30
            s_s[0] = 0.0

        s = jnp.sum(part_ref[...], axis=0, keepdims=True)
        logits = lax.dot_general(
            s, w_ref[...], (((1,), (1,)), ((), ())),
            preferred_element_type=jnp.float32,
        ) + b_ref[...]
        col = j * BLK + lax.broadcasted_iota(jnp.int32, (1, BLK), 1)
        logits = jnp.where(col < V, logits, -1e30)
        out_ref[...] = logits

        m_old = m_s[0]
        s_old = s_s[0]
        m_new = jnp.maximum(m_old, jnp.max(logits))
        s_new = s_old * jnp.exp(m_old - m_new) + jnp.sum(jnp.exp(logits - m_new))
        m_s[0] = m_new
        s_s[0] = s_new

        @pl.when(j == NB - 1)
        def _():
            lse_ref[0, 0] = m_new + jnp.log(s_new)

    return pl.pallas_call(
        body,
        grid=(NB,),
        in_specs=[
            pl.BlockSpec((NW, D), lambda j: (0, 0)),
            pl.BlockSpec((BLK, D), lambda j: (j, 0)),
            pl.BlockSpec((1, BLK), lambda j: (0, j)),
        ],
        out_specs=[
            pl.BlockSpec((1, BLK), lambda j: (0, j)),
            pl.BlockSpec((1, 1), lambda j: (0, 0), memory_space=pltpu.SMEM),
        ],
        out_shape=[
            jax.ShapeDtypeStruct((1, V), jnp.float32),
            jax.ShapeDtypeStruct((1, 1), jnp.float32),
        ],
        scratch_shapes=[
            pltpu.SMEM((1,), jnp.float32),
            pltpu.SMEM((1,), jnp.float32),
        ],
    )(partials, W, b2d)


def _tc_normalize(logits, lse):
    def body(log_ref, lse_ref, out_ref):
        out_ref[...] = log_ref[...] - lse_ref[0, 0]

    return pl.pallas_call(
        body,
        in_specs=[
            pl.BlockSpec(memory_space=pltpu.VMEM),
            pl.BlockSpec(memory_space=pltpu.SMEM),
        ],
        out_specs=pl.BlockSpec(memory_space=pltpu.VMEM),
        out_shape=jax.ShapeDtypeStruct((1, V), jnp.float32),
    )(logits, lse)


def kernel(inputs, emb_table, W, b):
    idx2d = inputs.astype(jnp.int32).reshape(NW * NCHUNK, CHUNK)
    partials = _sc_gather_sum(idx2d, emb_table)
    logits, lse = _tc_logits_lse(partials, W, b.reshape(1, V))
    return _tc_normalize(logits, lse)


# speedup vs baseline: 1.2937x; 1.0866x over previous
import functools

import jax
import jax.numpy as jnp
from jax import lax
from jax.experimental import pallas as pl
from jax.experimental.pallas import tpu as pltpu
from jax.experimental.pallas import tpu_sc as plsc

V = 100000
D = 128
CTX = 16384
NW = 32
ROWS_PER_W = CTX // NW
CHUNK = 128
NCHUNK = ROWS_PER_W // CHUNK
LANES = 16
NVEC = D // LANES

BLK = 8192
NB = (V + BLK - 1) // BLK


def _sc_gather_sum(idx2d, table):
    mesh = plsc.VectorSubcoreMesh(core_axis_name="c", subcore_axis_name="s")

    @functools.partial(
        pl.kernel,
        out_type=jax.ShapeDtypeStruct((NW, D), jnp.float32),
        mesh=mesh,
        scratch_types=[
            pltpu.VMEM((NCHUNK, CHUNK), jnp.int32),
            pltpu.VMEM((NCHUNK, CHUNK, D), jnp.float32),
            pltpu.VMEM((D,), jnp.float32),
            pltpu.SemaphoreType.DMA,
        ],
    )
    def k(idx_hbm, table_hbm, out_hbm, idx_v, rows_v, acc_v, sem):
        wid = lax.axis_index("s") * 2 + lax.axis_index("c")
        pltpu.sync_copy(idx_hbm.at[pl.ds(wid * NCHUNK, NCHUNK)], idx_v)
        copies = [
            pltpu.async_copy(table_hbm.at[idx_v.at[c]], rows_v.at[c], sem)
            for c in range(NCHUNK)
        ]

        def outer(acc, c):
            def body(i, acc):
                return tuple(
                    acc[j] + rows_v[c, i, pl.ds(j * LANES, LANES)]
                    for j in range(NVEC)
                )
            return lax.fori_loop(0, CHUNK, body, acc)

        acc = tuple(jnp.zeros((LANES,), jnp.float32) for _ in range(NVEC))
        for c in range(NCHUNK):
            copies[c].wait()
            acc = outer(acc, c)
        for j in range(NVEC):
            acc_v[pl.ds(j * LANES, LANES)] = acc[j]
        pltpu.sync_copy(acc_v, out_hbm.at[wid])

    return k(idx2d, table)


def _tc_log_probs(partials, W, b2d):
    def body(part_ref, w_ref, b_ref, out_ref, log_v, m_s, s_s):
        j = pl.program_id(0)

        @pl.when(j == 0)
        def _():
            m_s[0] = -1e30
            s_s[0] = 0.0

        s = jnp.sum(part_ref[...], axis=0, keepdims=True)
        logits = lax.dot_general(
            s, w_ref[...], (((1,), (1,)), ((), ())),
            preferred_element_type=jnp.float32,
        ) + b_ref[...]
        col = j * BLK + lax.broadcasted_iota(jnp.int32, (1, BLK), 1)
        logits = jnp.where(col < V, logits, -1e30)
        log_v[j] = logits

        m_old = m_s[0]
        s_old = s_s[0]
        m_new = jnp.maximum(m_old, jnp.max(logits))
        s_new = s_old * jnp.exp(m_old - m_new) + jnp.sum(jnp.exp(logits - m_new))
        m_s[0] = m_new
        s_s[0] = s_new

        @pl.when(j == NB - 1)
        def _():
            lse = m_new + jnp.log(s_new)
            for k in range(NB):
                width = min(BLK, V - k * BLK)
                out_ref[:, k * BLK:k * BLK + width] = (
                    log_v[k][:, :width] - lse
                )

    return pl.pallas_call(
        body,
        grid=(NB,),
        in_specs=[
            pl.BlockSpec((NW, D), lambda j: (0, 0)),
            pl.BlockSpec((BLK, D), lambda j: (j, 0)),
            pl.BlockSpec((1, BLK), lambda j: (0, j)),
        ],
        out_specs=pl.BlockSpec((1, V), lambda j: (0, 0)),
        out_shape=jax.ShapeDtypeStruct((1, V), jnp.float32),
        scratch_shapes=[
            pltpu.VMEM((NB, 1, BLK), jnp.float32),
            pltpu.SMEM((1,), jnp.float32),
            pltpu.SMEM((1,), jnp.float32),
        ],
    )(partials, W, b2d)


def kernel(inputs, emb_table, W, b):
    idx2d = inputs.astype(jnp.int32).reshape(NW * NCHUNK, CHUNK)
    partials = _sc_gather_sum(idx2d, emb_table)
    return _tc_log_probs(partials, W, b.reshape(1, V))


# speedup vs baseline: 1.3632x; 1.0538x over previous
import functools

import jax
import jax.numpy as jnp
from jax import lax
from jax.experimental import pallas as pl
from jax.experimental.pallas import tpu as pltpu
from jax.experimental.pallas import tpu_sc as plsc

V = 100000
D = 128
CTX = 16384
NW = 32
ROWS_PER_W = CTX // NW
CHUNK = 128
NCHUNK = ROWS_PER_W // CHUNK
LANES = 16
NVEC = D // LANES

BLK = 16384
NB = (V + BLK - 1) // BLK


def _sc_gather_sum(idx2d, table):
    mesh = plsc.VectorSubcoreMesh(core_axis_name="c", subcore_axis_name="s")

    @functools.partial(
        pl.kernel,
        out_type=jax.ShapeDtypeStruct((NW, D), jnp.float32),
        mesh=mesh,
        scratch_types=[
            pltpu.VMEM((NCHUNK, CHUNK), jnp.int32),
            pltpu.VMEM((NCHUNK, CHUNK, D), jnp.float32),
            pltpu.VMEM((D,), jnp.float32),
            pltpu.SemaphoreType.DMA,
        ],
    )
    def k(idx_hbm, table_hbm, out_hbm, idx_v, rows_v, acc_v, sem):
        wid = lax.axis_index("s") * 2 + lax.axis_index("c")
        pltpu.sync_copy(idx_hbm.at[pl.ds(wid * NCHUNK, NCHUNK)], idx_v)
        copies = [
            pltpu.async_copy(table_hbm.at[idx_v.at[c]], rows_v.at[c], sem)
            for c in range(NCHUNK)
        ]

        def outer(acc, c):
            def body(i, acc):
                return tuple(
                    acc[j] + rows_v[c, i, pl.ds(j * LANES, LANES)]
                    for j in range(NVEC)
                )
            return lax.fori_loop(0, CHUNK, body, acc, unroll=4)

        acc = tuple(jnp.zeros((LANES,), jnp.float32) for _ in range(NVEC))
        for c in range(NCHUNK):
            copies[c].wait()
            acc = outer(acc, c)
        for j in range(NVEC):
            acc_v[pl.ds(j * LANES, LANES)] = acc[j]
        pltpu.sync_copy(acc_v, out_hbm.at[wid])

    return k(idx2d, table)


def _tc_log_probs(partials, W, b2d):
    def body(part_ref, w_ref, b_ref, out_ref, log_v, m_s, s_s):
        j = pl.program_id(0)

        @pl.when(j == 0)
        def _():
            m_s[0] = -1e30
            s_s[0] = 0.0

        s = jnp.sum(part_ref[...], axis=0, keepdims=True)
        logits = lax.dot_general(
            s, w_ref[...], (((1,), (1,)), ((), ())),
            preferred_element_type=jnp.float32,
        ) + b_ref[...]
        col = j * BLK + lax.broadcasted_iota(jnp.int32, (1, BLK), 1)
        logits = jnp.where(col < V, logits, -1e30)
        log_v[j] = logits

        m_old = m_s[0]
        s_old = s_s[0]
        m_new = jnp.maximum(m_old, jnp.max(logits))
        s_new = s_old * jnp.exp(m_old - m_new) + jnp.sum(jnp.exp(logits - m_new))
        m_s[0] = m_new
        s_s[0] = s_new

        @pl.when(j == NB - 1)
        def _():
            lse = m_new + jnp.log(s_new)
            for k in range(NB):
                width = min(BLK, V - k * BLK)
                out_ref[:, k * BLK:k * BLK + width] = (
                    log_v[k][:, :width] - lse
                )

    return pl.pallas_call(
        body,
        grid=(NB,),
        in_specs=[
            pl.BlockSpec((NW, D), lambda j: (0, 0)),
            pl.BlockSpec((BLK, D), lambda j: (j, 0)),
            pl.BlockSpec((1, BLK), lambda j: (0, j)),
        ],
        out_specs=pl.BlockSpec((1, V), lambda j: (0, 0)),
        out_shape=jax.ShapeDtypeStruct((1, V), jnp.float32),
        scratch_shapes=[
            pltpu.VMEM((NB, 1, BLK), jnp.float32),
            pltpu.SMEM((1,), jnp.float32),
            pltpu.SMEM((1,), jnp.float32),
        ],
    )(partials, W, b2d)


def kernel(inputs, emb_table, W, b):
    idx2d = inputs.astype(jnp.int32).reshape(NW * NCHUNK, CHUNK)
    partials = _sc_gather_sum(idx2d, emb_table)
    return _tc_log_probs(partials, W, b.reshape(1, V))


# speedup vs baseline: 1.3789x; 1.0115x over previous
import functools

import jax
import jax.numpy as jnp
from jax import lax
from jax.experimental import pallas as pl
from jax.experimental.pallas import tpu as pltpu
from jax.experimental.pallas import tpu_sc as plsc

V = 100000
D = 128
CTX = 16384
NW = 32
ROWS_PER_W = CTX // NW
CHUNK = 128
NCHUNK = ROWS_PER_W // CHUNK
LANES = 16
NVEC = D // LANES

BLK = 25600
NB = (V + BLK - 1) // BLK


def _sc_gather_sum(idx2d, table):
    mesh = plsc.VectorSubcoreMesh(core_axis_name="c", subcore_axis_name="s")

    @functools.partial(
        pl.kernel,
        out_type=jax.ShapeDtypeStruct((NW, D), jnp.float32),
        mesh=mesh,
        scratch_types=[
            pltpu.VMEM((NCHUNK, CHUNK), jnp.int32),
            pltpu.VMEM((NCHUNK, CHUNK, D), jnp.float32),
            pltpu.VMEM((D,), jnp.float32),
            pltpu.SemaphoreType.DMA,
        ],
    )
    def k(idx_hbm, table_hbm, out_hbm, idx_v, rows_v, acc_v, sem):
        wid = lax.axis_index("s") * 2 + lax.axis_index("c")
        pltpu.sync_copy(idx_hbm.at[pl.ds(wid * NCHUNK, NCHUNK)], idx_v)
        copies = [
            pltpu.async_copy(table_hbm.at[idx_v.at[c]], rows_v.at[c], sem)
            for c in range(NCHUNK)
        ]

        def outer(acc, c):
            def body(i, acc):
                return tuple(
                    acc[j] + rows_v[c, i, pl.ds(j * LANES, LANES)]
                    for j in range(NVEC)
                )
            return lax.fori_loop(0, CHUNK, body, acc, unroll=4)

        acc = tuple(jnp.zeros((LANES,), jnp.float32) for _ in range(NVEC))
        for c in range(NCHUNK):
            copies[c].wait()
            acc = outer(acc, c)
        for j in range(NVEC):
            acc_v[pl.ds(j * LANES, LANES)] = acc[j]
        pltpu.sync_copy(acc_v, out_hbm.at[wid])

    return k(idx2d, table)


def _tc_log_probs(partials, W, b2d):
    def body(part_ref, w_ref, b_ref, out_ref, log_v, m_s, s_s):
        j = pl.program_id(0)

        @pl.when(j == 0)
        def _():
            m_s[0] = -1e30
            s_s[0] = 0.0

        s = jnp.sum(part_ref[...], axis=0, keepdims=True)
        logits = lax.dot_general(
            s, w_ref[...], (((1,), (1,)), ((), ())),
            preferred_element_type=jnp.float32,
        ) + b_ref[...]
        col = j * BLK + lax.broadcasted_iota(jnp.int32, (1, BLK), 1)
        logits = jnp.where(col < V, logits, -1e30)
        log_v[j] = logits

        m_old = m_s[0]
        s_old = s_s[0]
        m_new = jnp.maximum(m_old, jnp.max(logits))
        s_new = s_old * jnp.exp(m_old - m_new) + jnp.sum(jnp.exp(logits - m_new))
        m_s[0] = m_new
        s_s[0] = s_new

        @pl.when(j == NB - 1)
        def _():
            lse = m_new + jnp.log(s_new)
            for k in range(NB):
                width = min(BLK, V - k * BLK)
                out_ref[:, k * BLK:k * BLK + width] = (
                    log_v[k][:, :width] - lse
                )

    return pl.pallas_call(
        body,
        grid=(NB,),
        in_specs=[
            pl.BlockSpec((NW, D), lambda j: (0, 0)),
            pl.BlockSpec((BLK, D), lambda j: (j, 0)),
            pl.BlockSpec((1, BLK), lambda j: (0, j)),
        ],
        out_specs=pl.BlockSpec((1, V), lambda j: (0, 0)),
        out_shape=jax.ShapeDtypeStruct((1, V), jnp.float32),
        scratch_shapes=[
            pltpu.VMEM((NB, 1, BLK), jnp.float32),
            pltpu.SMEM((1,), jnp.float32),
            pltpu.SMEM((1,), jnp.float32),
        ],
    )(partials, W, b2d)


def kernel(inputs, emb_table, W, b):
    idx2d = inputs.astype(jnp.int32).reshape(NW * NCHUNK, CHUNK)
    partials = _sc_gather_sum(idx2d, emb_table)
    return _tc_log_probs(partials, W, b.reshape(1, V))
